# baseline (speedup 1.0000x reference)
import functools

import jax
import jax.numpy as jnp
from jax import lax
from jax.experimental import pallas as pl
from jax.experimental.pallas import tpu as pltpu
from jax.experimental.pallas import tpu_sc as plsc

N = 4096
CH = 8
NCHUNK = N // CH
NCORES = 2
NSUB = 16
NTILES = NCORES * NSUB
CPT = NCHUNK // NTILES
NPART = 4
CPP = CPT // NPART
GBUF = 80
PAD = GBUF * 16
NACC = 3
RING = 5
HIST = NCHUNK + 16

_mesh = plsc.VectorSubcoreMesh(core_axis_name="c", subcore_axis_name="s")


@functools.lru_cache(maxsize=None)
def _make_routing(nnz):
    tot_g = (nnz + 15) // 16
    gp = (tot_g + NSUB - 1) // NSUB
    nblk_h = (gp + GBUF - 1) // GBUF

    @functools.partial(
        pl.kernel,
        out_type=jax.ShapeDtypeStruct((NCHUNK + 16,), jnp.int32),
        mesh=_mesh,
        scratch_types=[
            pltpu.VMEM((PAD,), jnp.int32),
            pltpu.VMEM((HIST,), jnp.int32),
            pltpu.VMEM((NSUB, NCHUNK), jnp.int32),
            pltpu.VMEM((NCHUNK + 16,), jnp.int32),
            pltpu.VMEM_SHARED((NSUB, NCHUNK), jnp.int32),
        ],
        compiler_params=pltpu.CompilerParams(needs_layout_passes=False),
    )
    def routing(col_hbm, starts_hbm, c0, hist_v, allh_v, starts_v, sh_hist):
        sid = lax.axis_index("s")
        cc = lax.axis_index("c")

        def zh(i, _):
            hist_v[pl.ds(i * 16, 16)] = jnp.zeros((16,), jnp.int32)
            return 0
        lax.fori_loop(0, HIST // 16, zh, 0)

        g_lo = sid * gp
        g_hi = jnp.minimum(g_lo + gp, tot_g)
        ones = jnp.ones((16,), jnp.int32)

        def hblk(b, _):
            g = g_lo + b * GBUF
            off = pl.multiple_of(g * 16, 16)
            pltpu.sync_copy(col_hbm.at[pl.ds(off, PAD)], c0)
            nb = jnp.clip(g_hi - g, 0, GBUF)

            def hb(j, _):
                cv = c0[pl.ds(j * 16, 16)]
                plsc.addupdate_scatter(hist_v, [cv >> 3], ones)
                return 0
            lax.fori_loop(0, nb, hb, 0)
            return 0
        lax.fori_loop(0, nblk_h, hblk, 0)

        pltpu.sync_copy(hist_v.at[pl.ds(0, NCHUNK)], sh_hist.at[sid])
        plsc.subcore_barrier()

        @pl.when((sid == 0) & (cc == 0))
        def _():
            pltpu.sync_copy(sh_hist, allh_v)
            carry = jnp.zeros((16,), jnp.int32)
            for gi in range(NCHUNK // 16):
                tot = allh_v[0, pl.ds(gi * 16, 16)]
                for r in range(1, NSUB):
                    tot = tot + allh_v[r, pl.ds(gi * 16, 16)]
                inc = plsc.cumsum(tot)
                starts_v[pl.ds(gi * 16, 16)] = carry + inc - tot
                carry = jnp.full((16,), carry[15] + inc[15], jnp.int32)
            starts_v[pl.ds(NCHUNK, 16)] = jnp.full((16,), nnz, jnp.int32)
            pltpu.sync_copy(starts_v, starts_hbm)

    return routing


@functools.lru_cache(maxsize=None)
def _make_scatter(part):
    k_base = part * CPP * NTILES

    @functools.partial(
        pl.kernel,
        out_type=jax.ShapeDtypeStruct((N // 128, NCHUNK // NPART, CH, 128),
                                      jnp.float32),
        mesh=_mesh,
        scratch_types=[
            *[pltpu.VMEM((PAD,), jnp.int32) for _ in range(RING)],
            *[pltpu.VMEM((PAD,), jnp.int32) for _ in range(RING)],
            *[pltpu.VMEM((PAD,), jnp.float32) for _ in range(2)],
            *[pltpu.VMEM((N // 128, CH, 128), jnp.float32)
              for _ in range(NACC)],
            pltpu.VMEM((NCHUNK + 16,), jnp.int32),
            *[pltpu.SemaphoreType.DMA for _ in range(2 + NACC)],
        ],
        compiler_params=pltpu.CompilerParams(needs_layout_passes=False),
    )
    def scatter(row_hbm, col_hbm, val_hbm, starts_hbm, w_hbm,
                r0, r1, r2, r3, r4, c0, c1, c2, c3, c4, v0, v1,
                acc0, acc1, acc2, starts_v,
                ssem0, ssem1, osem0, osem1, osem2):
        rows = [r0, r1, r2, r3, r4]
        cols = [c0, c1, c2, c3, c4]
        vals = [v0, v1]
        ssem = [ssem0, ssem1]
        accs = [acc0, acc1, acc2]
        osem = [osem0, osem1, osem2]
        sid = lax.axis_index("s")
        wid = sid * NCORES + lax.axis_index("c")
        pltpu.sync_copy(starts_hbm, starts_v)

        def zero_dense(acc):
            def zb(i, _):
                acc[i >> 6, (i >> 3) & 7, pl.ds((i & 7) * 16, 16)] = (
                    jnp.zeros((16,), jnp.float32))
                return 0
            lax.fori_loop(0, CH * N // 16, zb, 0, unroll=8)

        def bounds(k):
            biv = jnp.full((16,), k, jnp.int32) + jnp.minimum(
                lax.iota(jnp.int32, 16), 1)
            bv = plsc.load_gather(starts_v, [biv])
            return bv[0], bv[1]

        def stage_pair(bi, si, g):
            off = pl.multiple_of(g * 16, 16)
            return [
                pltpu.make_async_copy(
                    row_hbm.at[pl.ds(off, PAD)], rows[bi], ssem[si]),
                pltpu.make_async_copy(
                    col_hbm.at[pl.ds(off, PAD)], cols[bi], ssem[si]),
                pltpu.make_async_copy(
                    val_hbm.at[pl.ds(off, PAD)], vals[si], ssem[si]),
            ]

        def do_groups(bi, si, acc, g_base, n_groups, s, e):
            def jb(j, _):
                rv = rows[bi][pl.ds(j * 16, 16)]
                cv = cols[bi][pl.ds(j * 16, 16)]
                vv = vals[si][pl.ds(j * 16, 16)]
                le = (g_base + j) * 16 + lax.iota(jnp.int32, 16)
                mk = (le >= s) & (le < e)
                plsc.addupdate_scatter(
                    acc, [rv >> 7, cv & (CH - 1), rv & 127], vv, mask=mk)
                return 0
            lax.fori_loop(0, n_groups, jb, 0)

        zeros16 = jnp.zeros((16,), jnp.float32)

        def zero_replay(bi, acc, g_base, n_groups, s, e):
            def jb(j, _):
                rv = rows[bi][pl.ds(j * 16, 16)]
                cv = cols[bi][pl.ds(j * 16, 16)]
                le = (g_base + j) * 16 + lax.iota(jnp.int32, 16)
                mk = (le >= s) & (le < e)
                plsc.store_scatter(
                    acc, [rv >> 7, cv & (CH - 1), rv & 127], zeros16,
                    mask=mk)
                return 0
            lax.fori_loop(0, n_groups, jb, 0)

        for a in accs:
            zero_dense(a)

        s_cur, e_cur = bounds(k_base + wid)
        for cp in stage_pair(0, 0, s_cur // 16):
            cp.start()

        hist_kk = {}
        for kk in range(CPP):
            k = k_base + kk * NTILES + wid
            k_loc = kk * NTILES + wid
            bi = kk % RING
            si = kk % 2
            ai = kk % NACC
            if kk + 1 < CPP:
                s_nxt, e_nxt = bounds(k + NTILES)
                for cp in stage_pair((kk + 1) % RING, 1 - si, s_nxt // 16):
                    cp.start()
            g0 = s_cur // 16
            g_end = (e_cur + 15) // 16
            nblk = (g_end - g0 + GBUF - 1) // GBUF
            for cp in stage_pair(bi, si, g0):
                cp.wait()
            if kk >= NACC:
                okk = kk - NACC
                o_bi, o_g0, o_nb0, o_s, o_e, o_nblk = hist_kk[okk]
                prev_loc = okk * NTILES + wid
                pltpu.make_async_copy(
                    accs[ai], w_hbm.at[:, prev_loc], osem[ai]).wait()

                @pl.when(o_nblk == 1)
                def _():
                    zero_replay(o_bi, accs[ai], o_g0, o_nb0, o_s, o_e)

                @pl.when(o_nblk > 1)
                def _():
                    zero_dense(accs[ai])

            nb0 = jnp.minimum(GBUF, g_end - g0)
            do_groups(bi, si, accs[ai], g0, nb0, s_cur, e_cur)
            hist_kk[kk] = (bi, g0, nb0, s_cur, e_cur, nblk)

            def extra(b, _):
                g = g0 + b * GBUF
                off = pl.multiple_of(g * 16, 16)
                pltpu.sync_copy(row_hbm.at[pl.ds(off, PAD)], rows[bi])
                pltpu.sync_copy(col_hbm.at[pl.ds(off, PAD)], cols[bi])
                pltpu.sync_copy(val_hbm.at[pl.ds(off, PAD)], vals[si])
                do_groups(bi, si, accs[ai], g,
                          jnp.minimum(GBUF, g_end - g), s_cur, e_cur)
                return 0
            lax.fori_loop(1, nblk, extra, 0)

            pltpu.make_async_copy(
                accs[ai], w_hbm.at[:, k_loc], osem[ai]).start()
            if kk + 1 < CPP:
                s_cur, e_cur = s_nxt, e_nxt

        for kk in range(CPP - NACC, CPP):
            ai = kk % NACC
            k_loc = kk * NTILES + wid
            pltpu.make_async_copy(
                accs[ai], w_hbm.at[:, k_loc], osem[ai]).wait()

    return scatter


_NB = 256


def _mm_body(x_ref, w_ref, o_ref):
    acc = jnp.zeros((x_ref.shape[0], _NB), jnp.float32)
    for r in range(N // 128):
        xr = x_ref[:, r * 128:(r + 1) * 128].astype(jnp.bfloat16)
        wr = w_ref[r].reshape(_NB, 128).astype(jnp.bfloat16)
        acc = acc + lax.dot_general(
            xr, wr, (((1,), (1,)), ((), ())),
            preferred_element_type=jnp.float32)
    o_ref[...] = jnp.maximum(acc, 0.0)


def kernel(inputs, values, row_idx, col_idx):
    B, n = inputs.shape
    nnz = values.shape[0]

    xpad = PAD + 16
    row_p = jnp.concatenate([row_idx, jnp.zeros((xpad,), jnp.int32)])
    col_p = jnp.concatenate([col_idx, jnp.full((xpad,), N, jnp.int32)])
    val_p = jnp.concatenate([values, jnp.zeros((xpad,), jnp.float32)])

    starts = _make_routing(nnz)(col_p)

    npc = NCHUNK // NPART
    outs = []
    for part in range(NPART):
        w4 = _make_scatter(part)(row_p, col_p, val_p, starts)
        outs.append(pl.pallas_call(
            _mm_body,
            grid=(npc // (_NB // CH),),
            in_specs=[
                pl.BlockSpec((B, N), lambda i: (0, 0)),
                pl.BlockSpec((N // 128, _NB // CH, CH, 128),
                             lambda i: (0, i, 0, 0)),
            ],
            out_specs=pl.BlockSpec((B, _NB), lambda i: (0, i)),
            out_shape=jax.ShapeDtypeStruct((B, N // NPART), jnp.float32),
        )(inputs, w4))
    return jnp.concatenate(outs, axis=1)


# speedup vs baseline: 1.1302x; 1.1302x over previous
import functools

import jax
import jax.numpy as jnp
from jax import lax
from jax.experimental import pallas as pl
from jax.experimental.pallas import tpu as pltpu
from jax.experimental.pallas import tpu_sc as plsc

N = 4096
CH = 8
NCHUNK = N // CH
NCORES = 2
NSUB = 16
NTILES = NCORES * NSUB
CPT = NCHUNK // NTILES
NPART = 2
CPP = CPT // NPART
GBUF = 80
PAD = GBUF * 16
NACC = 3
RING = 5
HIST = NCHUNK + 16

_mesh = plsc.VectorSubcoreMesh(core_axis_name="c", subcore_axis_name="s")


@functools.lru_cache(maxsize=None)
def _make_routing(nnz):
    tot_g = (nnz + 15) // 16
    gp = (tot_g + NSUB - 1) // NSUB
    nblk_h = (gp + GBUF - 1) // GBUF

    @functools.partial(
        pl.kernel,
        out_type=jax.ShapeDtypeStruct((NCHUNK + 16,), jnp.int32),
        mesh=_mesh,
        scratch_types=[
            pltpu.VMEM((PAD,), jnp.int32),
            pltpu.VMEM((HIST,), jnp.int32),
            pltpu.VMEM((NSUB, NCHUNK), jnp.int32),
            pltpu.VMEM((NCHUNK + 16,), jnp.int32),
            pltpu.VMEM_SHARED((NSUB, NCHUNK), jnp.int32),
        ],
        compiler_params=pltpu.CompilerParams(needs_layout_passes=False),
    )
    def routing(col_hbm, starts_hbm, c0, hist_v, allh_v, starts_v, sh_hist):
        sid = lax.axis_index("s")
        cc = lax.axis_index("c")

        def zh(i, _):
            hist_v[pl.ds(i * 16, 16)] = jnp.zeros((16,), jnp.int32)
            return 0
        lax.fori_loop(0, HIST // 16, zh, 0)

        g_lo = sid * gp
        g_hi = jnp.minimum(g_lo + gp, tot_g)
        ones = jnp.ones((16,), jnp.int32)

        def hblk(b, _):
            g = g_lo + b * GBUF
            off = pl.multiple_of(g * 16, 16)
            pltpu.sync_copy(col_hbm.at[pl.ds(off, PAD)], c0)
            nb = jnp.clip(g_hi - g, 0, GBUF)

            def hb(j, _):
                cv = c0[pl.ds(j * 16, 16)]
                plsc.addupdate_scatter(hist_v, [cv >> 3], ones)
                return 0
            lax.fori_loop(0, nb, hb, 0)
            return 0
        lax.fori_loop(0, nblk_h, hblk, 0)

        pltpu.sync_copy(hist_v.at[pl.ds(0, NCHUNK)], sh_hist.at[sid])
        plsc.subcore_barrier()

        @pl.when((sid == 0) & (cc == 0))
        def _():
            pltpu.sync_copy(sh_hist, allh_v)
            carry = jnp.zeros((16,), jnp.int32)
            for gi in range(NCHUNK // 16):
                tot = allh_v[0, pl.ds(gi * 16, 16)]
                for r in range(1, NSUB):
                    tot = tot + allh_v[r, pl.ds(gi * 16, 16)]
                inc = plsc.cumsum(tot)
                starts_v[pl.ds(gi * 16, 16)] = carry + inc - tot
                carry = jnp.full((16,), carry[15] + inc[15], jnp.int32)
            starts_v[pl.ds(NCHUNK, 16)] = jnp.full((16,), nnz, jnp.int32)
            pltpu.sync_copy(starts_v, starts_hbm)

    return routing


@functools.lru_cache(maxsize=None)
def _make_scatter(part):
    k_base = part * CPP * NTILES

    @functools.partial(
        pl.kernel,
        out_type=jax.ShapeDtypeStruct((N // 128, NCHUNK // NPART, CH, 128),
                                      jnp.float32),
        mesh=_mesh,
        scratch_types=[
            *[pltpu.VMEM((PAD,), jnp.int32) for _ in range(RING)],
            *[pltpu.VMEM((PAD,), jnp.int32) for _ in range(RING)],
            *[pltpu.VMEM((PAD,), jnp.float32) for _ in range(2)],
            *[pltpu.VMEM((N // 128, CH, 128), jnp.float32)
              for _ in range(NACC)],
            pltpu.VMEM((NCHUNK + 16,), jnp.int32),
            *[pltpu.SemaphoreType.DMA for _ in range(2 + NACC)],
        ],
        compiler_params=pltpu.CompilerParams(needs_layout_passes=False),
    )
    def scatter(row_hbm, col_hbm, val_hbm, starts_hbm, w_hbm,
                r0, r1, r2, r3, r4, c0, c1, c2, c3, c4, v0, v1,
                acc0, acc1, acc2, starts_v,
                ssem0, ssem1, osem0, osem1, osem2):
        rows = [r0, r1, r2, r3, r4]
        cols = [c0, c1, c2, c3, c4]
        vals = [v0, v1]
        ssem = [ssem0, ssem1]
        accs = [acc0, acc1, acc2]
        osem = [osem0, osem1, osem2]
        sid = lax.axis_index("s")
        wid = sid * NCORES + lax.axis_index("c")
        pltpu.sync_copy(starts_hbm, starts_v)

        def zero_dense(acc):
            def zb(i, _):
                acc[i >> 6, (i >> 3) & 7, pl.ds((i & 7) * 16, 16)] = (
                    jnp.zeros((16,), jnp.float32))
                return 0
            lax.fori_loop(0, CH * N // 16, zb, 0, unroll=8)

        def bounds(k):
            biv = jnp.full((16,), k, jnp.int32) + jnp.minimum(
                lax.iota(jnp.int32, 16), 1)
            bv = plsc.load_gather(starts_v, [biv])
            return bv[0], bv[1]

        def stage_pair(bi, si, g):
            off = pl.multiple_of(g * 16, 16)
            return [
                pltpu.make_async_copy(
                    row_hbm.at[pl.ds(off, PAD)], rows[bi], ssem[si]),
                pltpu.make_async_copy(
                    col_hbm.at[pl.ds(off, PAD)], cols[bi], ssem[si]),
                pltpu.make_async_copy(
                    val_hbm.at[pl.ds(off, PAD)], vals[si], ssem[si]),
            ]

        def do_groups(bi, si, acc, g_base, n_groups, s, e):
            def jb(j, _):
                rv = rows[bi][pl.ds(j * 16, 16)]
                cv = cols[bi][pl.ds(j * 16, 16)]
                vv = vals[si][pl.ds(j * 16, 16)]
                le = (g_base + j) * 16 + lax.iota(jnp.int32, 16)
                mk = (le >= s) & (le < e)
                plsc.addupdate_scatter(
                    acc, [rv >> 7, cv & (CH - 1), rv & 127], vv, mask=mk)
                return 0
            lax.fori_loop(0, n_groups, jb, 0)

        zeros16 = jnp.zeros((16,), jnp.float32)

        def zero_replay(bi, acc, g_base, n_groups, s, e):
            def jb(j, _):
                rv = rows[bi][pl.ds(j * 16, 16)]
                cv = cols[bi][pl.ds(j * 16, 16)]
                le = (g_base + j) * 16 + lax.iota(jnp.int32, 16)
                mk = (le >= s) & (le < e)
                plsc.store_scatter(
                    acc, [rv >> 7, cv & (CH - 1), rv & 127], zeros16,
                    mask=mk)
                return 0
            lax.fori_loop(0, n_groups, jb, 0)

        for a in accs:
            zero_dense(a)

        s_cur, e_cur = bounds(k_base + wid)
        for cp in stage_pair(0, 0, s_cur // 16):
            cp.start()

        hist_kk = {}
        for kk in range(CPP):
            k = k_base + kk * NTILES + wid
            k_loc = kk * NTILES + wid
            bi = kk % RING
            si = kk % 2
            ai = kk % NACC
            if kk + 1 < CPP:
                s_nxt, e_nxt = bounds(k + NTILES)
                for cp in stage_pair((kk + 1) % RING, 1 - si, s_nxt // 16):
                    cp.start()
            g0 = s_cur // 16
            g_end = (e_cur + 15) // 16
            nblk = (g_end - g0 + GBUF - 1) // GBUF
            for cp in stage_pair(bi, si, g0):
                cp.wait()
            if kk >= NACC:
                okk = kk - NACC
                o_bi, o_g0, o_nb0, o_s, o_e, o_nblk = hist_kk[okk]
                prev_loc = okk * NTILES + wid
                pltpu.make_async_copy(
                    accs[ai], w_hbm.at[:, prev_loc], osem[ai]).wait()

                @pl.when(o_nblk == 1)
                def _():
                    zero_replay(o_bi, accs[ai], o_g0, o_nb0, o_s, o_e)

                @pl.when(o_nblk > 1)
                def _():
                    zero_dense(accs[ai])

            nb0 = jnp.minimum(GBUF, g_end - g0)
            do_groups(bi, si, accs[ai], g0, nb0, s_cur, e_cur)
            hist_kk[kk] = (bi, g0, nb0, s_cur, e_cur, nblk)

            def extra(b, _):
                g = g0 + b * GBUF
                off = pl.multiple_of(g * 16, 16)
                pltpu.sync_copy(row_hbm.at[pl.ds(off, PAD)], rows[bi])
                pltpu.sync_copy(col_hbm.at[pl.ds(off, PAD)], cols[bi])
                pltpu.sync_copy(val_hbm.at[pl.ds(off, PAD)], vals[si])
                do_groups(bi, si, accs[ai], g,
                          jnp.minimum(GBUF, g_end - g), s_cur, e_cur)
                return 0
            lax.fori_loop(1, nblk, extra, 0)

            pltpu.make_async_copy(
                accs[ai], w_hbm.at[:, k_loc], osem[ai]).start()
            if kk + 1 < CPP:
                s_cur, e_cur = s_nxt, e_nxt

        for kk in range(CPP - NACC, CPP):
            ai = kk % NACC
            k_loc = kk * NTILES + wid
            pltpu.make_async_copy(
                accs[ai], w_hbm.at[:, k_loc], osem[ai]).wait()

    return scatter


_NB = 256


def _mm_body(x_ref, w_ref, o_ref):
    acc = jnp.zeros((x_ref.shape[0], _NB), jnp.float32)
    for r in range(N // 128):
        xr = x_ref[:, r * 128:(r + 1) * 128].astype(jnp.bfloat16)
        wr = w_ref[r].reshape(_NB, 128).astype(jnp.bfloat16)
        acc = acc + lax.dot_general(
            xr, wr, (((1,), (1,)), ((), ())),
            preferred_element_type=jnp.float32)
    o_ref[...] = jnp.maximum(acc, 0.0)


def kernel(inputs, values, row_idx, col_idx):
    B, n = inputs.shape
    nnz = values.shape[0]

    xpad = PAD + 16
    row_p = jnp.concatenate([row_idx, jnp.zeros((xpad,), jnp.int32)])
    col_p = jnp.concatenate([col_idx, jnp.full((xpad,), N, jnp.int32)])
    val_p = jnp.concatenate([values, jnp.zeros((xpad,), jnp.float32)])

    starts = _make_routing(nnz)(col_p)

    npc = NCHUNK // NPART
    outs = []
    for part in range(NPART):
        w4 = _make_scatter(part)(row_p, col_p, val_p, starts)
        outs.append(pl.pallas_call(
            _mm_body,
            grid=(npc // (_NB // CH),),
            in_specs=[
                pl.BlockSpec((B, N), lambda i: (0, 0)),
                pl.BlockSpec((N // 128, _NB // CH, CH, 128),
                             lambda i: (0, i, 0, 0)),
            ],
            out_specs=pl.BlockSpec((B, _NB), lambda i: (0, i)),
            out_shape=jax.ShapeDtypeStruct((B, N // NPART), jnp.float32),
        )(inputs, w4))
    return jnp.concatenate(outs, axis=1)


# speedup vs baseline: 1.2702x; 1.1239x over previous
import functools

import jax
import jax.numpy as jnp
from jax import lax
from jax.experimental import pallas as pl
from jax.experimental.pallas import tpu as pltpu
from jax.experimental.pallas import tpu_sc as plsc

N = 4096
CH = 8
NCHUNK = N // CH
NCORES = 2
NSUB = 16
NTILES = NCORES * NSUB
CPT = NCHUNK // NTILES
GBUF = 80
PAD = GBUF * 16
NACC = 3
RING = 5
HIST = NCHUNK + 16


@functools.lru_cache(maxsize=None)
def _make_scatter(nnz):
    mesh = plsc.VectorSubcoreMesh(core_axis_name="c", subcore_axis_name="s")

    tot_g = (nnz + 15) // 16
    gp = (tot_g + NSUB - 1) // NSUB
    nblk_h = (gp + GBUF - 1) // GBUF

    @functools.partial(
        pl.kernel,
        out_type=jax.ShapeDtypeStruct((N // 128, NCHUNK, CH, 128),
                                      jnp.float32),
        mesh=mesh,
        scratch_types=[
            *[pltpu.VMEM((PAD,), jnp.int32) for _ in range(RING)],
            *[pltpu.VMEM((PAD,), jnp.int32) for _ in range(RING)],
            *[pltpu.VMEM((PAD,), jnp.float32) for _ in range(2)],
            *[pltpu.VMEM((N // 128, CH, 128), jnp.float32)
              for _ in range(NACC)],
            pltpu.VMEM((HIST,), jnp.int32),
            pltpu.VMEM((NSUB, NCHUNK), jnp.int32),
            pltpu.VMEM((NCHUNK + 16,), jnp.int32),
            pltpu.VMEM_SHARED((NSUB, NCHUNK), jnp.int32),
            *[pltpu.SemaphoreType.DMA for _ in range(2 + NACC)],
        ],
        compiler_params=pltpu.CompilerParams(needs_layout_passes=False),
    )
    def scatter(row_hbm, col_hbm, val_hbm, w_hbm,
                r0, r1, r2, r3, r4, c0, c1, c2, c3, c4, v0, v1,
                acc0, acc1, acc2, hist_v, allh_v, starts_v, sh_hist,
                ssem0, ssem1, osem0, osem1, osem2):
        rows = [r0, r1, r2, r3, r4]
        cols = [c0, c1, c2, c3, c4]
        vals = [v0, v1]
        ssem = [ssem0, ssem1]
        accs = [acc0, acc1, acc2]
        osem = [osem0, osem1, osem2]
        sid = lax.axis_index("s")
        wid = sid * NCORES + lax.axis_index("c")

        def zh(i, _):
            hist_v[pl.ds(i * 16, 16)] = jnp.zeros((16,), jnp.int32)
            return 0
        lax.fori_loop(0, HIST // 16, zh, 0)

        g_lo = sid * gp
        g_hi = jnp.minimum(g_lo + gp, tot_g)
        ones = jnp.ones((16,), jnp.int32)

        def hblk(b, _):
            g = g_lo + b * GBUF
            off = pl.multiple_of(g * 16, 16)
            pltpu.sync_copy(col_hbm.at[pl.ds(off, PAD)], c0)
            nb = jnp.clip(g_hi - g, 0, GBUF)

            def hb(j, _):
                cv = c0[pl.ds(j * 16, 16)]
                plsc.addupdate_scatter(hist_v, [cv >> 3], ones)
                return 0
            lax.fori_loop(0, nb, hb, 0)
            return 0
        lax.fori_loop(0, nblk_h, hblk, 0)

        pltpu.sync_copy(hist_v.at[pl.ds(0, NCHUNK)], sh_hist.at[sid])
        plsc.subcore_barrier()
        pltpu.sync_copy(sh_hist, allh_v)

        carry = jnp.zeros((16,), jnp.int32)
        for gi in range(NCHUNK // 16):
            tot = allh_v[0, pl.ds(gi * 16, 16)]
            for r in range(1, NSUB):
                tot = tot + allh_v[r, pl.ds(gi * 16, 16)]
            inc = plsc.cumsum(tot)
            starts_v[pl.ds(gi * 16, 16)] = carry + inc - tot
            carry = jnp.full((16,), carry[15] + inc[15], jnp.int32)
        starts_v[pl.ds(NCHUNK, 16)] = jnp.full((16,), nnz, jnp.int32)

        def zero_dense(acc):
            def zb(i, _):
                acc[i >> 6, (i >> 3) & 7, pl.ds((i & 7) * 16, 16)] = (
                    jnp.zeros((16,), jnp.float32))
                return 0
            lax.fori_loop(0, CH * N // 16, zb, 0, unroll=8)

        def bounds(k):
            biv = jnp.full((16,), k, jnp.int32) + jnp.minimum(
                lax.iota(jnp.int32, 16), 1)
            bv = plsc.load_gather(starts_v, [biv])
            return bv[0], bv[1]

        def stage_pair(bi, si, g):
            off = pl.multiple_of(g * 16, 16)
            return [
                pltpu.make_async_copy(
                    row_hbm.at[pl.ds(off, PAD)], rows[bi], ssem[si]),
                pltpu.make_async_copy(
                    col_hbm.at[pl.ds(off, PAD)], cols[bi], ssem[si]),
                pltpu.make_async_copy(
                    val_hbm.at[pl.ds(off, PAD)], vals[si], ssem[si]),
            ]

        def do_groups(bi, si, acc, g_base, n_groups, s, e):
            def jb(j, _):
                rv = rows[bi][pl.ds(j * 16, 16)]
                cv = cols[bi][pl.ds(j * 16, 16)]
                vv = vals[si][pl.ds(j * 16, 16)]
                le = (g_base + j) * 16 + lax.iota(jnp.int32, 16)
                mk = (le >= s) & (le < e)
                plsc.addupdate_scatter(
                    acc, [rv >> 7, cv & (CH - 1), rv & 127], vv, mask=mk)
                return 0
            lax.fori_loop(0, n_groups, jb, 0)

        zeros16 = jnp.zeros((16,), jnp.float32)

        def zero_replay(bi, acc, g_base, n_groups, s, e):
            def jb(j, _):
                rv = rows[bi][pl.ds(j * 16, 16)]
                cv = cols[bi][pl.ds(j * 16, 16)]
                le = (g_base + j) * 16 + lax.iota(jnp.int32, 16)
                mk = (le >= s) & (le < e)
                plsc.store_scatter(
                    acc, [rv >> 7, cv & (CH - 1), rv & 127], zeros16,
                    mask=mk)
                return 0
            lax.fori_loop(0, n_groups, jb, 0)

        for a in accs:
            zero_dense(a)

        s_cur, e_cur = bounds(wid)
        for cp in stage_pair(0, 0, s_cur // 16):
            cp.start()

        hist_kk = {}
        for kk in range(CPT):
            k = kk * NTILES + wid
            bi = kk % RING
            si = kk % 2
            ai = kk % NACC
            if kk + 1 < CPT:
                s_nxt, e_nxt = bounds(k + NTILES)
                for cp in stage_pair((kk + 1) % RING, 1 - si, s_nxt // 16):
                    cp.start()
            g0 = s_cur // 16
            g_end = (e_cur + 15) // 16
            nblk = (g_end - g0 + GBUF - 1) // GBUF
            for cp in stage_pair(bi, si, g0):
                cp.wait()
            if kk >= NACC:
                okk = kk - NACC
                o_bi, o_g0, o_nb0, o_s, o_e, o_nblk = hist_kk[okk]
                prev_k = okk * NTILES + wid
                pltpu.make_async_copy(
                    accs[ai], w_hbm.at[:, prev_k], osem[ai]).wait()

                @pl.when(o_nblk == 1)
                def _():
                    zero_replay(o_bi, accs[ai], o_g0, o_nb0, o_s, o_e)

                @pl.when(o_nblk > 1)
                def _():
                    zero_dense(accs[ai])

            nb0 = jnp.minimum(GBUF, g_end - g0)
            do_groups(bi, si, accs[ai], g0, nb0, s_cur, e_cur)
            hist_kk[kk] = (bi, g0, nb0, s_cur, e_cur, nblk)

            def extra(b, _):
                g = g0 + b * GBUF
                off = pl.multiple_of(g * 16, 16)
                pltpu.sync_copy(row_hbm.at[pl.ds(off, PAD)], rows[bi])
                pltpu.sync_copy(col_hbm.at[pl.ds(off, PAD)], cols[bi])
                pltpu.sync_copy(val_hbm.at[pl.ds(off, PAD)], vals[si])
                do_groups(bi, si, accs[ai], g,
                          jnp.minimum(GBUF, g_end - g), s_cur, e_cur)
                return 0
            lax.fori_loop(1, nblk, extra, 0)

            pltpu.make_async_copy(
                accs[ai], w_hbm.at[:, k], osem[ai]).start()
            if kk + 1 < CPT:
                s_cur, e_cur = s_nxt, e_nxt

        for kk in range(CPT - NACC, CPT):
            ai = kk % NACC
            k = kk * NTILES + wid
            pltpu.make_async_copy(
                accs[ai], w_hbm.at[:, k], osem[ai]).wait()

    return scatter


_NB = 512


def _mm_body(x_ref, w_ref, o_ref):
    acc = jnp.zeros((x_ref.shape[0], _NB), jnp.float32)
    for r in range(N // 128):
        xr = x_ref[:, r * 128:(r + 1) * 128].astype(jnp.bfloat16)
        wr = w_ref[r].reshape(_NB, 128).astype(jnp.bfloat16)
        acc = acc + lax.dot_general(
            xr, wr, (((1,), (1,)), ((), ())),
            preferred_element_type=jnp.float32)
    o_ref[...] = jnp.maximum(acc, 0.0)


def kernel(inputs, values, row_idx, col_idx):
    B, n = inputs.shape
    nnz = values.shape[0]

    xpad = PAD + 16
    row_p = jnp.concatenate([row_idx, jnp.zeros((xpad,), jnp.int32)])
    col_p = jnp.concatenate([col_idx, jnp.full((xpad,), N, jnp.int32)])
    val_p = jnp.concatenate([values, jnp.zeros((xpad,), jnp.float32)])

    w4 = _make_scatter(nnz)(row_p, col_p, val_p)

    out = pl.pallas_call(
        _mm_body,
        grid=(N // _NB,),
        in_specs=[
            pl.BlockSpec((B, N), lambda i: (0, 0)),
            pl.BlockSpec((N // 128, _NB // CH, CH, 128),
                         lambda i: (0, i, 0, 0)),
        ],
        out_specs=pl.BlockSpec((B, _NB), lambda i: (0, i)),
        out_shape=jax.ShapeDtypeStruct((B, N), jnp.float32),
    )(inputs, w4)
    return out


# speedup vs baseline: 1.2743x; 1.0032x over previous
import functools

import jax
import jax.numpy as jnp
from jax import lax
from jax.experimental import pallas as pl
from jax.experimental.pallas import tpu as pltpu
from jax.experimental.pallas import tpu_sc as plsc

N = 4096
CH = 8
NCHUNK = N // CH
NCORES = 2
NSUB = 16
NTILES = NCORES * NSUB
CPT = NCHUNK // NTILES
GBUF = 80
PAD = GBUF * 16
NACC = 3
RING = 5
HIST = NCHUNK + 16


@functools.lru_cache(maxsize=None)
def _make_scatter(nnz):
    mesh = plsc.VectorSubcoreMesh(core_axis_name="c", subcore_axis_name="s")

    tot_g = (nnz + 15) // 16
    gp = (tot_g + NSUB - 1) // NSUB
    nblk_h = (gp + GBUF - 1) // GBUF

    @functools.partial(
        pl.kernel,
        out_type=jax.ShapeDtypeStruct((N // 128, NCHUNK, CH, 128),
                                      jnp.float32),
        mesh=mesh,
        scratch_types=[
            *[pltpu.VMEM((PAD,), jnp.int32) for _ in range(RING)],
            *[pltpu.VMEM((PAD,), jnp.int32) for _ in range(RING)],
            *[pltpu.VMEM((PAD,), jnp.float32) for _ in range(2)],
            *[pltpu.VMEM((N // 128, CH, 128), jnp.float32)
              for _ in range(NACC)],
            pltpu.VMEM((HIST,), jnp.int32),
            pltpu.VMEM((NSUB, NCHUNK), jnp.int32),
            pltpu.VMEM((NCHUNK + 16,), jnp.int32),
            pltpu.VMEM_SHARED((NSUB, NCHUNK), jnp.int32),
            *[pltpu.SemaphoreType.DMA for _ in range(2 + NACC)],
        ],
        compiler_params=pltpu.CompilerParams(needs_layout_passes=False),
    )
    def scatter(row_hbm, col_hbm, val_hbm, w_hbm,
                r0, r1, r2, r3, r4, c0, c1, c2, c3, c4, v0, v1,
                acc0, acc1, acc2, hist_v, allh_v, starts_v, sh_hist,
                ssem0, ssem1, osem0, osem1, osem2):
        rows = [r0, r1, r2, r3, r4]
        cols = [c0, c1, c2, c3, c4]
        vals = [v0, v1]
        ssem = [ssem0, ssem1]
        accs = [acc0, acc1, acc2]
        osem = [osem0, osem1, osem2]
        sid = lax.axis_index("s")
        wid = sid * NCORES + lax.axis_index("c")

        def zh(i, _):
            hist_v[pl.ds(i * 16, 16)] = jnp.zeros((16,), jnp.int32)
            return 0
        lax.fori_loop(0, HIST // 16, zh, 0)

        g_lo = sid * gp
        g_hi = jnp.minimum(g_lo + gp, tot_g)
        ones = jnp.ones((16,), jnp.int32)

        def hblk(b, _):
            g = g_lo + b * GBUF
            off = pl.multiple_of(g * 16, 16)
            pltpu.sync_copy(col_hbm.at[pl.ds(off, PAD)], c0)
            nb = jnp.clip(g_hi - g, 0, GBUF)

            def hb(j, _):
                cv = c0[pl.ds(j * 16, 16)]
                plsc.addupdate_scatter(hist_v, [cv >> 3], ones)
                return 0
            lax.fori_loop(0, nb, hb, 0)
            return 0
        lax.fori_loop(0, nblk_h, hblk, 0)

        pltpu.sync_copy(hist_v.at[pl.ds(0, NCHUNK)], sh_hist.at[sid])
        plsc.subcore_barrier()
        pltpu.sync_copy(sh_hist, allh_v)

        carry = jnp.zeros((16,), jnp.int32)
        for gi in range(NCHUNK // 16):
            tot = allh_v[0, pl.ds(gi * 16, 16)]
            for r in range(1, NSUB):
                tot = tot + allh_v[r, pl.ds(gi * 16, 16)]
            inc = plsc.cumsum(tot)
            starts_v[pl.ds(gi * 16, 16)] = carry + inc - tot
            carry = jnp.full((16,), carry[15] + inc[15], jnp.int32)
        starts_v[pl.ds(NCHUNK, 16)] = jnp.full((16,), nnz, jnp.int32)

        def zero_dense(acc):
            def zb(i, _):
                acc[i >> 6, (i >> 3) & 7, pl.ds((i & 7) * 16, 16)] = (
                    jnp.zeros((16,), jnp.float32))
                return 0
            lax.fori_loop(0, CH * N // 16, zb, 0, unroll=8)

        def bounds(k):
            biv = jnp.full((16,), k, jnp.int32) + jnp.minimum(
                lax.iota(jnp.int32, 16), 1)
            bv = plsc.load_gather(starts_v, [biv])
            return bv[0], bv[1]

        def stage_pair(bi, si, g):
            off = pl.multiple_of(g * 16, 16)
            return [
                pltpu.make_async_copy(
                    row_hbm.at[pl.ds(off, PAD)], rows[bi], ssem[si]),
                pltpu.make_async_copy(
                    col_hbm.at[pl.ds(off, PAD)], cols[bi], ssem[si]),
                pltpu.make_async_copy(
                    val_hbm.at[pl.ds(off, PAD)], vals[si], ssem[si]),
            ]

        def do_groups(bi, si, acc, g_base, n_groups, s, e):
            def jb(j, _):
                rv = rows[bi][pl.ds(j * 16, 16)]
                cv = cols[bi][pl.ds(j * 16, 16)]
                vv = vals[si][pl.ds(j * 16, 16)]
                le = (g_base + j) * 16 + lax.iota(jnp.int32, 16)
                mk = (le >= s) & (le < e)
                plsc.addupdate_scatter(
                    acc, [rv >> 7, cv & (CH - 1), rv & 127], vv, mask=mk)
                return 0
            lax.fori_loop(0, n_groups, jb, 0)

        zeros16 = jnp.zeros((16,), jnp.float32)

        def zero_replay(bi, acc, g_base, n_groups, s, e):
            def jb(j, _):
                rv = rows[bi][pl.ds(j * 16, 16)]
                cv = cols[bi][pl.ds(j * 16, 16)]
                le = (g_base + j) * 16 + lax.iota(jnp.int32, 16)
                mk = (le >= s) & (le < e)
                plsc.store_scatter(
                    acc, [rv >> 7, cv & (CH - 1), rv & 127], zeros16,
                    mask=mk)
                return 0
            lax.fori_loop(0, n_groups, jb, 0)

        for a in accs:
            zero_dense(a)

        s_cur, e_cur = bounds(wid)
        for cp in stage_pair(0, 0, s_cur // 16):
            cp.start()

        hist_kk = {}
        for kk in range(CPT):
            k = kk * NTILES + wid
            bi = kk % RING
            si = kk % 2
            ai = kk % NACC
            if kk + 1 < CPT:
                s_nxt, e_nxt = bounds(k + NTILES)
                for cp in stage_pair((kk + 1) % RING, 1 - si, s_nxt // 16):
                    cp.start()
            g0 = s_cur // 16
            g_end = (e_cur + 15) // 16
            nblk = (g_end - g0 + GBUF - 1) // GBUF
            for cp in stage_pair(bi, si, g0):
                cp.wait()
            if kk >= NACC:
                okk = kk - NACC
                o_bi, o_g0, o_nb0, o_s, o_e, o_nblk = hist_kk[okk]
                prev_k = okk * NTILES + wid
                pltpu.make_async_copy(
                    accs[ai], w_hbm.at[:, prev_k], osem[ai]).wait()

                @pl.when(o_nblk == 1)
                def _():
                    zero_replay(o_bi, accs[ai], o_g0, o_nb0, o_s, o_e)

                @pl.when(o_nblk > 1)
                def _():
                    zero_dense(accs[ai])

            nb0 = jnp.minimum(GBUF, g_end - g0)
            do_groups(bi, si, accs[ai], g0, nb0, s_cur, e_cur)
            hist_kk[kk] = (bi, g0, nb0, s_cur, e_cur, nblk)

            def extra(b, _):
                g = g0 + b * GBUF
                off = pl.multiple_of(g * 16, 16)
                pltpu.sync_copy(row_hbm.at[pl.ds(off, PAD)], rows[bi])
                pltpu.sync_copy(col_hbm.at[pl.ds(off, PAD)], cols[bi])
                pltpu.sync_copy(val_hbm.at[pl.ds(off, PAD)], vals[si])
                do_groups(bi, si, accs[ai], g,
                          jnp.minimum(GBUF, g_end - g), s_cur, e_cur)
                return 0
            lax.fori_loop(1, nblk, extra, 0)

            pltpu.make_async_copy(
                accs[ai], w_hbm.at[:, k], osem[ai]).start()
            if kk + 1 < CPT:
                s_cur, e_cur = s_nxt, e_nxt

        for kk in range(CPT - NACC, CPT):
            ai = kk % NACC
            k = kk * NTILES + wid
            pltpu.make_async_copy(
                accs[ai], w_hbm.at[:, k], osem[ai]).wait()

    return scatter


_NB = 1024


def _mm_body(x_ref, w_ref, o_ref):
    acc = jnp.zeros((x_ref.shape[0], _NB), jnp.float32)
    for r in range(N // 128):
        xr = x_ref[:, r * 128:(r + 1) * 128].astype(jnp.bfloat16)
        wr = w_ref[r].reshape(_NB, 128).astype(jnp.bfloat16)
        acc = acc + lax.dot_general(
            xr, wr, (((1,), (1,)), ((), ())),
            preferred_element_type=jnp.float32)
    o_ref[...] = jnp.maximum(acc, 0.0)


def kernel(inputs, values, row_idx, col_idx):
    B, n = inputs.shape
    nnz = values.shape[0]

    xpad = PAD + 16
    row_p = jnp.concatenate([row_idx, jnp.zeros((xpad,), jnp.int32)])
    col_p = jnp.concatenate([col_idx, jnp.full((xpad,), N, jnp.int32)])
    val_p = jnp.concatenate([values, jnp.zeros((xpad,), jnp.float32)])

    w4 = _make_scatter(nnz)(row_p, col_p, val_p)

    out = pl.pallas_call(
        _mm_body,
        grid=(N // _NB,),
        in_specs=[
            pl.BlockSpec((B, N), lambda i: (0, 0)),
            pl.BlockSpec((N // 128, _NB // CH, CH, 128),
                         lambda i: (0, i, 0, 0)),
        ],
        out_specs=pl.BlockSpec((B, _NB), lambda i: (0, i)),
        out_shape=jax.ShapeDtypeStruct((B, N), jnp.float32),
    )(inputs, w4)
    return out
